# baseline (speedup 1.0000x reference)
import functools

import jax
import jax.numpy as jnp
from jax import lax
from jax.experimental import pallas as pl
from jax.experimental.pallas import tpu as pltpu
from jax.experimental.pallas import tpu_sc as plsc

N = 10000
E = 320000
F_IN = 128
DIM_H = 64
G = 64

NC = 2
NS = 16
NW = NC * NS
EPW = E // NW
WIN = 128
NWIN = EPW // WIN
TAIL = EPW - NWIN * WIN
DWIN = 128
DNWIN = EPW // DWIN
DTAIL = EPW - DNWIN * DWIN
RING = 8
MAIN = ((NWIN - 1) // RING) * RING
RPT = 624
NREM = N - NS * RPT
NZW = RPT // WIN
ZREM = RPT - NZW * WIN

FP = DIM_H

BLK = 1000
NBLK = N // BLK

_mesh = plsc.VectorSubcoreMesh(core_axis_name="c", subcore_axis_name="s")


@functools.partial(
    pl.kernel,
    out_type=[jax.ShapeDtypeStruct((N,), jnp.float32),
              jax.ShapeDtypeStruct((N,), jnp.float32)],
    mesh=_mesh,
    scratch_types=[
        pltpu.VMEM((DNWIN, DWIN), jnp.int32),
        pltpu.VMEM((1, DTAIL), jnp.int32),
        pltpu.VMEM((DWIN,), jnp.float32),
        pltpu.VMEM((RPT,), jnp.float32),
        pltpu.VMEM_SHARED((N,), jnp.float32),
    ],
)
def _deg(colm_hbm, colt_hbm, onesm_hbm, zeros1_hbm,
         out0, out1, colm, colt, onesm, stage, dacc):
    cid = lax.axis_index("c")
    sid = lax.axis_index("s")
    w = cid * NS + sid
    pltpu.sync_copy(colm_hbm.at[w], colm)
    pltpu.sync_copy(colt_hbm.at[w], colt)
    pltpu.sync_copy(onesm_hbm, onesm)
    pltpu.sync_copy(zeros1_hbm, stage)
    pltpu.sync_copy(stage, dacc.at[pl.ds(sid * RPT, RPT)])

    @pl.when(sid == NS - 1)
    def _():
        pltpu.sync_copy(stage.at[pl.ds(0, NREM)],
                        dacc.at[pl.ds(NS * RPT, NREM)])

    plsc.subcore_barrier()
    @pl.loop(0, DNWIN)
    def _(j):
        pltpu.sync_copy(onesm, dacc.at[colm.at[j]], add=True)

    pltpu.sync_copy(onesm.at[pl.ds(0, DTAIL)], dacc.at[colt.at[0]], add=True)
    plsc.subcore_barrier()

    pltpu.sync_copy(dacc.at[pl.ds(sid * RPT, RPT)], stage)

    @pl.when(cid == 0)
    def _():
        pltpu.sync_copy(stage, out0.at[pl.ds(sid * RPT, RPT)])

    @pl.when(cid == 1)
    def _():
        pltpu.sync_copy(stage, out1.at[pl.ds(sid * RPT, RPT)])

    @pl.when(sid == NS - 1)
    def _():
        pltpu.sync_copy(dacc.at[pl.ds(NS * RPT, NREM)],
                        stage.at[pl.ds(0, NREM)])

        @pl.when(cid == 0)
        def _():
            pltpu.sync_copy(stage.at[pl.ds(0, NREM)],
                            out0.at[pl.ds(NS * RPT, NREM)])

        @pl.when(cid == 1)
        def _():
            pltpu.sync_copy(stage.at[pl.ds(0, NREM)],
                            out1.at[pl.ds(NS * RPT, NREM)])


@functools.partial(
    pl.kernel,
    out_type=[jax.ShapeDtypeStruct((N, FP), jnp.float32),
              jax.ShapeDtypeStruct((N, FP), jnp.float32)],
    mesh=_mesh,
    scratch_types=[
        pltpu.VMEM((NWIN, WIN), jnp.int32),
        pltpu.VMEM((NWIN, WIN), jnp.int32),
        pltpu.VMEM((1, TAIL), jnp.int32),
        pltpu.VMEM((1, TAIL), jnp.int32),
    ] + [pltpu.VMEM((WIN, FP), jnp.float32)] * RING + [
        pltpu.VMEM((TAIL, FP), jnp.float32),
        pltpu.VMEM_SHARED((N, FP), jnp.float32),
    ] + [pltpu.SemaphoreType.DMA] * (2 * RING),
    compiler_params=pltpu.CompilerParams(use_tc_tiling_on_sc=False),
)
def _mp(xs_hbm, rowm_hbm, colm_hbm, rowt_hbm, colt_hbm, zeros_hbm,
        out0, out1, rowm, colm, rowt, colt, *rest):
    bufs = list(rest[:RING])
    buft = rest[RING]
    acc = rest[RING + 1]
    gs = list(rest[RING + 2:RING + 2 + RING])
    ss = list(rest[RING + 2 + RING:])
    cid = lax.axis_index("c")
    sid = lax.axis_index("s")
    w = cid * NS + sid
    pltpu.sync_copy(rowm_hbm.at[w], rowm)
    pltpu.sync_copy(colm_hbm.at[w], colm)
    pltpu.sync_copy(rowt_hbm.at[w], rowt)
    pltpu.sync_copy(colt_hbm.at[w], colt)

    base = sid * RPT
    pltpu.sync_copy(zeros_hbm, bufs[0])
    zcp = []
    for k in range(NZW):
        zcp.append(pltpu.async_copy(
            bufs[0], acc.at[pl.ds(base + k * WIN, WIN)], gs[k]))
    zr = pltpu.async_copy(bufs[0].at[pl.ds(0, ZREM)],
                          acc.at[pl.ds(base + NZW * WIN, ZREM)], gs[NZW])
    for c in zcp:
        c.wait()
    zr.wait()

    @pl.when(sid == NS - 1)
    def _():
        pltpu.sync_copy(bufs[0].at[pl.ds(0, NREM)],
                        acc.at[pl.ds(NS * RPT, NREM)])

    plsc.subcore_barrier()

    for b in range(RING):
        pltpu.async_copy(xs_hbm.at[rowm.at[b]], bufs[b], gs[b])

    @pl.loop(0, MAIN, step=RING)
    def _(j):
        for b in range(RING):
            pltpu.make_async_copy(xs_hbm.at[rowm.at[0]], bufs[b],
                                  gs[b]).wait()
            pltpu.async_copy(bufs[b], acc.at[colm.at[j + b]], ss[b],
                             add=True)
        for b in range(RING):
            pltpu.make_async_copy(bufs[b], acc.at[colm.at[0]],
                                  ss[b]).wait()
            w2 = j + RING + b

            @pl.when(w2 < NWIN)
            def _():
                pltpu.async_copy(xs_hbm.at[rowm.at[w2]], bufs[b], gs[b])

    ecp = []
    for b in range(NWIN - MAIN):
        pltpu.make_async_copy(xs_hbm.at[rowm.at[0]], bufs[b], gs[b]).wait()
        ecp.append(pltpu.async_copy(bufs[b], acc.at[colm.at[MAIN + b]],
                                    ss[b], add=True))
    pltpu.sync_copy(xs_hbm.at[rowt.at[0]], buft)
    pltpu.sync_copy(buft, acc.at[colt.at[0]], add=True)
    for c in ecp:
        c.wait()
    plsc.subcore_barrier()

    def _copy_out(out_ref):
        icp = []
        for k in range(NZW):
            icp.append(pltpu.async_copy(acc.at[pl.ds(base + k * WIN, WIN)],
                                        bufs[k], gs[k]))
        icp.append(pltpu.async_copy(
            acc.at[pl.ds(base + NZW * WIN, ZREM)],
            bufs[NZW].at[pl.ds(0, ZREM)], gs[NZW]))
        ocp = []
        for k in range(NZW):
            icp[k].wait()
            ocp.append(pltpu.async_copy(
                bufs[k], out_ref.at[pl.ds(base + k * WIN, WIN)], ss[k]))
        icp[NZW].wait()
        ocp.append(pltpu.async_copy(
            bufs[NZW].at[pl.ds(0, ZREM)],
            out_ref.at[pl.ds(base + NZW * WIN, ZREM)], ss[NZW]))

        @pl.when(sid == NS - 1)
        def _():
            pltpu.sync_copy(acc.at[pl.ds(NS * RPT, NREM)],
                            bufs[NZW + 1].at[pl.ds(0, NREM)])
            pltpu.sync_copy(bufs[NZW + 1].at[pl.ds(0, NREM)],
                            out_ref.at[pl.ds(NS * RPT, NREM)])

        for c in ocp:
            c.wait()

    @pl.when(cid == 0)
    def _():
        _copy_out(out0)

    @pl.when(cid == 1)
    def _():
        _copy_out(out1)


def _k1m_body(x_ref, w0_ref, xw_ref):
    xw_ref[...] = jnp.dot(x_ref[...], w0_ref[...],
                          preferred_element_type=jnp.float32)


def _k1m(x, W0):
    return pl.pallas_call(
        _k1m_body,
        grid=(NBLK,),
        in_specs=[
            pl.BlockSpec((BLK, F_IN), lambda i: (i, 0)),
            pl.BlockSpec((F_IN, DIM_H), lambda i: (0, 0)),
        ],
        out_specs=pl.BlockSpec((BLK, FP), lambda i: (i, 0)),
        out_shape=jax.ShapeDtypeStruct((N, FP), jnp.float32),
    )(x, W0)


def _k1s_body(xw_ref, d0_ref, d1_ref, xs_ref):
    dinv = lax.rsqrt(d0_ref[...] + d1_ref[...] + 1.0)
    xs_ref[...] = dinv * xw_ref[...]


def _k1s(xw, d0, d1):
    return pl.pallas_call(
        _k1s_body,
        grid=(NBLK,),
        in_specs=[
            pl.BlockSpec((BLK, FP), lambda i: (i, 0)),
            pl.BlockSpec((BLK, 1), lambda i: (i, 0)),
            pl.BlockSpec((BLK, 1), lambda i: (i, 0)),
        ],
        out_specs=pl.BlockSpec((BLK, FP), lambda i: (i, 0)),
        out_shape=jax.ShapeDtypeStruct((N, FP), jnp.float32),
    )(xw, d0, d1)


def _pool_accumulate(p_ref, h, bt_ref):
    bcol = bt_ref[0]
    g_lo = bt_ref[0, 0, 0]
    g_hi = bt_ref[0, BLK - 1, 0]

    def body(g, carry):
        mask = bcol == g
        hm = jnp.where(mask, h, 0.0)
        p_ref[0, pl.ds(g, 1), :] += jnp.sum(hm, axis=0, keepdims=True)
        p_ref[1, pl.ds(g, 1), :] = jnp.maximum(
            p_ref[1, pl.ds(g, 1), :], jnp.max(hm, axis=0, keepdims=True))
        p_ref[2, pl.ds(g, 1), :] += jnp.sum(mask.astype(jnp.float32))
        return carry

    lax.fori_loop(g_lo, g_hi + 1, body, 0)


def _k2_body(q0_ref, q1_ref, xs0_ref, d0_ref, d1_ref, b0_ref, w1_ref, bt_ref,
             xs1_ref, p1_ref):
    i = pl.program_id(0)
    dinv = lax.rsqrt(d0_ref[...] + d1_ref[...] + 1.0)
    h = jnp.maximum(
        dinv * (q0_ref[:, :DIM_H] + q1_ref[:, :DIM_H] + xs0_ref[:, :DIM_H])
        + b0_ref[...], 0.0)
    xs1_ref[...] = dinv * jnp.dot(h, w1_ref[...],
                                  preferred_element_type=jnp.float32)

    @pl.when(i == 0)
    def _():
        p1_ref[...] = jnp.zeros_like(p1_ref)

    _pool_accumulate(p1_ref, h, bt_ref)


def _k2(q0, q1, xs0, d0, d1, b0, W1, bt):
    return pl.pallas_call(
        _k2_body,
        grid=(NBLK,),
        in_specs=[
            pl.BlockSpec((BLK, FP), lambda i: (i, 0)),
            pl.BlockSpec((BLK, FP), lambda i: (i, 0)),
            pl.BlockSpec((BLK, FP), lambda i: (i, 0)),
            pl.BlockSpec((BLK, 1), lambda i: (i, 0)),
            pl.BlockSpec((BLK, 1), lambda i: (i, 0)),
            pl.BlockSpec((1, DIM_H), lambda i: (0, 0)),
            pl.BlockSpec((DIM_H, DIM_H), lambda i: (0, 0)),
            pl.BlockSpec((1, BLK, 1), lambda i: (i, 0, 0)),
        ],
        out_specs=[
            pl.BlockSpec((BLK, FP), lambda i: (i, 0)),
            pl.BlockSpec((3, G, DIM_H), lambda i: (0, 0, 0)),
        ],
        out_shape=[
            jax.ShapeDtypeStruct((N, FP), jnp.float32),
            jax.ShapeDtypeStruct((3, G, DIM_H), jnp.float32),
        ],
    )(q0, q1, xs0, d0, d1, b0, W1, bt)


def _k3_body(r0_ref, r1_ref, xs1_ref, d0_ref, d1_ref, b1_ref, bt_ref, p1_ref,
             l1w_ref, l1b_ref, l2w_ref, l2b_ref, sig_ref, z_ref, p2_ref):
    i = pl.program_id(0)
    dinv = lax.rsqrt(d0_ref[...] + d1_ref[...] + 1.0)
    h = jnp.maximum(
        dinv * (r0_ref[:, :DIM_H] + r1_ref[:, :DIM_H] + xs1_ref[:, :DIM_H])
        + b1_ref[...], 0.0)

    @pl.when(i == 0)
    def _():
        p2_ref[...] = jnp.zeros_like(p2_ref)

    _pool_accumulate(p2_ref, h, bt_ref)

    @pl.when(i == NBLK - 1)
    def _():
        s1, m1, c1 = p1_ref[0], p1_ref[1], p1_ref[2]
        s2, m2, c2 = p2_ref[0], p2_ref[1], p2_ref[2]
        mean1 = s1 / jnp.maximum(c1, 1.0)
        mean2 = s2 / jnp.maximum(c2, 1.0)
        hc = jnp.concatenate([s1, m1, mean1, s2, m2, mean2], axis=1)
        z1 = jnp.maximum(
            jnp.dot(hc, l1w_ref[...], preferred_element_type=jnp.float32)
            + l1b_ref[...], 0.0)
        z2 = jnp.dot(z1, l2w_ref[...],
                     preferred_element_type=jnp.float32) + l2b_ref[...]
        z_ref[...] = z2
        sig_ref[...] = 1.0 / (1.0 + jnp.exp(-z2))


def _k3(r0, r1, xs1, d0, d1, b1, bt, P1, l1w, l1b, l2w, l2b):
    RD = DIM_H * 2 * 3
    return pl.pallas_call(
        _k3_body,
        grid=(NBLK,),
        in_specs=[
            pl.BlockSpec((BLK, FP), lambda i: (i, 0)),
            pl.BlockSpec((BLK, FP), lambda i: (i, 0)),
            pl.BlockSpec((BLK, FP), lambda i: (i, 0)),
            pl.BlockSpec((BLK, 1), lambda i: (i, 0)),
            pl.BlockSpec((BLK, 1), lambda i: (i, 0)),
            pl.BlockSpec((1, DIM_H), lambda i: (0, 0)),
            pl.BlockSpec((1, BLK, 1), lambda i: (i, 0, 0)),
            pl.BlockSpec((3, G, DIM_H), lambda i: (0, 0, 0)),
            pl.BlockSpec((RD, RD), lambda i: (0, 0)),
            pl.BlockSpec((1, RD), lambda i: (0, 0)),
            pl.BlockSpec((RD, 1), lambda i: (0, 0)),
            pl.BlockSpec((1, 1), lambda i: (0, 0)),
        ],
        out_specs=[
            pl.BlockSpec((G, 1), lambda i: (0, 0)),
            pl.BlockSpec((G, 1), lambda i: (0, 0)),
        ],
        out_shape=[
            jax.ShapeDtypeStruct((G, 1), jnp.float32),
            jax.ShapeDtypeStruct((G, 1), jnp.float32),
        ],
        scratch_shapes=[pltpu.VMEM((3, G, DIM_H), jnp.float32)],
    )(r0, r1, xs1, d0, d1, b1, bt, P1, l1w, l1b, l2w, l2b)


def kernel(x, edge_index, batch, W0, b0, W1, b1, lin1_W, lin1_b, lin2_W,
           lin2_b):
    row = edge_index[0].reshape(NW, EPW)
    col = edge_index[1].reshape(NW, EPW)
    rowm = row[:, :NWIN * WIN].reshape(NW, NWIN, WIN)
    rowt = row[:, NWIN * WIN:].reshape(NW, 1, TAIL)
    colm = col[:, :NWIN * WIN].reshape(NW, NWIN, WIN)
    colt = col[:, NWIN * WIN:].reshape(NW, 1, TAIL)
    dcolm = col[:, :DNWIN * DWIN].reshape(NW, DNWIN, DWIN)
    dcolt = col[:, DNWIN * DWIN:].reshape(NW, 1, DTAIL)

    onesm = jnp.ones((DWIN,), jnp.float32)
    zeros1 = jnp.zeros((RPT,), jnp.float32)
    zeros64 = jnp.zeros((WIN, FP), jnp.float32)
    bt = batch.reshape(NBLK, BLK, 1)

    xw = _k1m(x, W0)
    d0, d1 = _deg(dcolm, dcolt, onesm, zeros1)
    d0 = d0.reshape(N, 1)
    d1 = d1.reshape(N, 1)
    xs0 = _k1s(xw, d0, d1)
    q0, q1 = _mp(xs0, rowm, colm, rowt, colt, zeros64)
    xs1, P1 = _k2(q0, q1, xs0, d0, d1, b0.reshape(1, DIM_H), W1, bt)
    r0, r1 = _mp(xs1, rowm, colm, rowt, colt, zeros64)
    sig, z = _k3(r0, r1, xs1, d0, d1, b1.reshape(1, DIM_H), bt, P1,
                 lin1_W, lin1_b.reshape(1, -1), lin2_W, lin2_b.reshape(1, 1))
    return (sig, z)


# speedup vs baseline: 1.0083x; 1.0083x over previous
import functools

import jax
import jax.numpy as jnp
from jax import lax
from jax.experimental import pallas as pl
from jax.experimental.pallas import tpu as pltpu
from jax.experimental.pallas import tpu_sc as plsc

N = 10000
E = 320000
F_IN = 128
DIM_H = 64
G = 64

NC = 2
NS = 16
NW = NC * NS
EPW = E // NW
WIN = 128
NWIN = EPW // WIN
TAIL = EPW - NWIN * WIN
DWIN = 128
DNWIN = EPW // DWIN
DTAIL = EPW - DNWIN * DWIN
RING = 8
MAIN = ((NWIN - 1) // RING) * RING
RPT = 624
NREM = N - NS * RPT
NZW = RPT // WIN
ZREM = RPT - NZW * WIN

FP = DIM_H

BLK = 1000
NBLK = N // BLK

_mesh = plsc.VectorSubcoreMesh(core_axis_name="c", subcore_axis_name="s")


@functools.partial(
    pl.kernel,
    out_type=[jax.ShapeDtypeStruct((N,), jnp.float32),
              jax.ShapeDtypeStruct((N,), jnp.float32)],
    mesh=_mesh,
    scratch_types=[
        pltpu.VMEM((DNWIN, DWIN), jnp.int32),
        pltpu.VMEM((1, DTAIL), jnp.int32),
        pltpu.VMEM((DWIN,), jnp.float32),
        pltpu.VMEM((RPT,), jnp.float32),
        pltpu.VMEM_SHARED((N,), jnp.float32),
    ],
)
def _deg(colm_hbm, colt_hbm, onesm_hbm, zeros1_hbm,
         out0, out1, colm, colt, onesm, stage, dacc):
    cid = lax.axis_index("c")
    sid = lax.axis_index("s")
    w = cid * NS + sid
    pltpu.sync_copy(colm_hbm.at[w], colm)
    pltpu.sync_copy(colt_hbm.at[w], colt)
    pltpu.sync_copy(onesm_hbm, onesm)
    pltpu.sync_copy(zeros1_hbm, stage)
    pltpu.sync_copy(stage, dacc.at[pl.ds(sid * RPT, RPT)])

    @pl.when(sid == NS - 1)
    def _():
        pltpu.sync_copy(stage.at[pl.ds(0, NREM)],
                        dacc.at[pl.ds(NS * RPT, NREM)])

    plsc.subcore_barrier()
    @pl.loop(0, DNWIN)
    def _(j):
        pltpu.sync_copy(onesm, dacc.at[colm.at[j]], add=True)

    pltpu.sync_copy(onesm.at[pl.ds(0, DTAIL)], dacc.at[colt.at[0]], add=True)
    plsc.subcore_barrier()

    pltpu.sync_copy(dacc.at[pl.ds(sid * RPT, RPT)], stage)

    @pl.when(cid == 0)
    def _():
        pltpu.sync_copy(stage, out0.at[pl.ds(sid * RPT, RPT)])

    @pl.when(cid == 1)
    def _():
        pltpu.sync_copy(stage, out1.at[pl.ds(sid * RPT, RPT)])

    @pl.when(sid == NS - 1)
    def _():
        pltpu.sync_copy(dacc.at[pl.ds(NS * RPT, NREM)],
                        stage.at[pl.ds(0, NREM)])

        @pl.when(cid == 0)
        def _():
            pltpu.sync_copy(stage.at[pl.ds(0, NREM)],
                            out0.at[pl.ds(NS * RPT, NREM)])

        @pl.when(cid == 1)
        def _():
            pltpu.sync_copy(stage.at[pl.ds(0, NREM)],
                            out1.at[pl.ds(NS * RPT, NREM)])


@functools.partial(
    pl.kernel,
    out_type=[jax.ShapeDtypeStruct((N, FP), jnp.float32),
              jax.ShapeDtypeStruct((N, FP), jnp.float32)],
    mesh=_mesh,
    scratch_types=[
        pltpu.VMEM((NWIN, WIN), jnp.int32),
        pltpu.VMEM((NWIN, WIN), jnp.int32),
        pltpu.VMEM((1, TAIL), jnp.int32),
        pltpu.VMEM((1, TAIL), jnp.int32),
    ] + [pltpu.VMEM((WIN, FP), jnp.float32)] * RING + [
        pltpu.VMEM((TAIL, FP), jnp.float32),
        pltpu.VMEM_SHARED((N, FP), jnp.float32),
    ] + [pltpu.SemaphoreType.DMA] * (2 * RING),
    compiler_params=pltpu.CompilerParams(use_tc_tiling_on_sc=False),
)
def _mp(xs_hbm, rowm_hbm, colm_hbm, rowt_hbm, colt_hbm, zeros_hbm,
        out0, out1, rowm, colm, rowt, colt, *rest):
    bufs = list(rest[:RING])
    buft = rest[RING]
    acc = rest[RING + 1]
    gs = list(rest[RING + 2:RING + 2 + RING])
    ss = list(rest[RING + 2 + RING:])
    cid = lax.axis_index("c")
    sid = lax.axis_index("s")
    w = cid * NS + sid
    pltpu.sync_copy(rowm_hbm.at[w], rowm)
    pltpu.sync_copy(colm_hbm.at[w], colm)
    pltpu.sync_copy(rowt_hbm.at[w], rowt)
    pltpu.sync_copy(colt_hbm.at[w], colt)

    base = sid * RPT
    pltpu.sync_copy(zeros_hbm, bufs[0])
    zcp = []
    for k in range(NZW):
        zcp.append(pltpu.async_copy(
            bufs[0], acc.at[pl.ds(base + k * WIN, WIN)], gs[k]))
    zr = pltpu.async_copy(bufs[0].at[pl.ds(0, ZREM)],
                          acc.at[pl.ds(base + NZW * WIN, ZREM)], gs[NZW])
    for c in zcp:
        c.wait()
    zr.wait()

    @pl.when(sid == NS - 1)
    def _():
        pltpu.sync_copy(bufs[0].at[pl.ds(0, NREM)],
                        acc.at[pl.ds(NS * RPT, NREM)])

    plsc.subcore_barrier()

    for b in range(RING):
        pltpu.async_copy(xs_hbm.at[rowm.at[b]], bufs[b], gs[b])

    @pl.loop(0, MAIN, step=RING)
    def _(j):
        for b in range(RING):
            pltpu.make_async_copy(xs_hbm.at[rowm.at[0]], bufs[b],
                                  gs[b]).wait()
            pltpu.async_copy(bufs[b], acc.at[colm.at[j + b]], ss[b],
                             add=True)
        for b in range(RING):
            pltpu.make_async_copy(bufs[b], acc.at[colm.at[0]],
                                  ss[b]).wait()
            w2 = j + RING + b

            @pl.when(w2 < NWIN)
            def _():
                pltpu.async_copy(xs_hbm.at[rowm.at[w2]], bufs[b], gs[b])

    ecp = []
    for b in range(NWIN - MAIN):
        pltpu.make_async_copy(xs_hbm.at[rowm.at[0]], bufs[b], gs[b]).wait()
        ecp.append(pltpu.async_copy(bufs[b], acc.at[colm.at[MAIN + b]],
                                    ss[b], add=True))
    pltpu.sync_copy(xs_hbm.at[rowt.at[0]], buft)
    pltpu.sync_copy(buft, acc.at[colt.at[0]], add=True)
    for c in ecp:
        c.wait()
    plsc.subcore_barrier()

    def _copy_out(out_ref):
        icp = []
        for k in range(NZW):
            icp.append(pltpu.async_copy(acc.at[pl.ds(base + k * WIN, WIN)],
                                        bufs[k], gs[k]))
        icp.append(pltpu.async_copy(
            acc.at[pl.ds(base + NZW * WIN, ZREM)],
            bufs[NZW].at[pl.ds(0, ZREM)], gs[NZW]))
        ocp = []
        for k in range(NZW):
            icp[k].wait()
            ocp.append(pltpu.async_copy(
                bufs[k], out_ref.at[pl.ds(base + k * WIN, WIN)], ss[k]))
        icp[NZW].wait()
        ocp.append(pltpu.async_copy(
            bufs[NZW].at[pl.ds(0, ZREM)],
            out_ref.at[pl.ds(base + NZW * WIN, ZREM)], ss[NZW]))

        @pl.when(sid == NS - 1)
        def _():
            pltpu.sync_copy(acc.at[pl.ds(NS * RPT, NREM)],
                            bufs[NZW + 1].at[pl.ds(0, NREM)])
            pltpu.sync_copy(bufs[NZW + 1].at[pl.ds(0, NREM)],
                            out_ref.at[pl.ds(NS * RPT, NREM)])

        for c in ocp:
            c.wait()

    @pl.when(cid == 0)
    def _():
        _copy_out(out0)

    @pl.when(cid == 1)
    def _():
        _copy_out(out1)


def _k1_body(x_ref, w0_ref, d0_ref, d1_ref, xs_ref):
    dinv = lax.rsqrt(d0_ref[...] + d1_ref[...] + 1.0)
    xw = jnp.dot(x_ref[...], w0_ref[...], preferred_element_type=jnp.float32)
    xs_ref[...] = dinv * xw


def _k1(x, W0, d0, d1):
    return pl.pallas_call(
        _k1_body,
        grid=(NBLK,),
        in_specs=[
            pl.BlockSpec((BLK, F_IN), lambda i: (i, 0)),
            pl.BlockSpec((F_IN, DIM_H), lambda i: (0, 0)),
            pl.BlockSpec((BLK, 1), lambda i: (i, 0)),
            pl.BlockSpec((BLK, 1), lambda i: (i, 0)),
        ],
        out_specs=pl.BlockSpec((BLK, FP), lambda i: (i, 0)),
        out_shape=jax.ShapeDtypeStruct((N, FP), jnp.float32),
    )(x, W0, d0, d1)


def _pool_accumulate(p_ref, h, bt_ref):
    bcol = bt_ref[0]
    g_lo = bt_ref[0, 0, 0]
    g_hi = bt_ref[0, BLK - 1, 0]

    def body(g, carry):
        mask = bcol == g
        hm = jnp.where(mask, h, 0.0)
        p_ref[0, pl.ds(g, 1), :] += jnp.sum(hm, axis=0, keepdims=True)
        p_ref[1, pl.ds(g, 1), :] = jnp.maximum(
            p_ref[1, pl.ds(g, 1), :], jnp.max(hm, axis=0, keepdims=True))
        p_ref[2, pl.ds(g, 1), :] += jnp.sum(mask.astype(jnp.float32))
        return carry

    lax.fori_loop(g_lo, g_hi + 1, body, 0)


def _k2_body(q0_ref, q1_ref, xs0_ref, d0_ref, d1_ref, b0_ref, w1_ref, bt_ref,
             xs1_ref, p1_ref):
    i = pl.program_id(0)
    dinv = lax.rsqrt(d0_ref[...] + d1_ref[...] + 1.0)
    h = jnp.maximum(
        dinv * (q0_ref[:, :DIM_H] + q1_ref[:, :DIM_H] + xs0_ref[:, :DIM_H])
        + b0_ref[...], 0.0)
    xs1_ref[...] = dinv * jnp.dot(h, w1_ref[...],
                                  preferred_element_type=jnp.float32)

    @pl.when(i == 0)
    def _():
        p1_ref[...] = jnp.zeros_like(p1_ref)

    _pool_accumulate(p1_ref, h, bt_ref)


def _k2(q0, q1, xs0, d0, d1, b0, W1, bt):
    return pl.pallas_call(
        _k2_body,
        grid=(NBLK,),
        in_specs=[
            pl.BlockSpec((BLK, FP), lambda i: (i, 0)),
            pl.BlockSpec((BLK, FP), lambda i: (i, 0)),
            pl.BlockSpec((BLK, FP), lambda i: (i, 0)),
            pl.BlockSpec((BLK, 1), lambda i: (i, 0)),
            pl.BlockSpec((BLK, 1), lambda i: (i, 0)),
            pl.BlockSpec((1, DIM_H), lambda i: (0, 0)),
            pl.BlockSpec((DIM_H, DIM_H), lambda i: (0, 0)),
            pl.BlockSpec((1, BLK, 1), lambda i: (i, 0, 0)),
        ],
        out_specs=[
            pl.BlockSpec((BLK, FP), lambda i: (i, 0)),
            pl.BlockSpec((3, G, DIM_H), lambda i: (0, 0, 0)),
        ],
        out_shape=[
            jax.ShapeDtypeStruct((N, FP), jnp.float32),
            jax.ShapeDtypeStruct((3, G, DIM_H), jnp.float32),
        ],
    )(q0, q1, xs0, d0, d1, b0, W1, bt)


def _k3_body(r0_ref, r1_ref, xs1_ref, d0_ref, d1_ref, b1_ref, bt_ref, p1_ref,
             l1w_ref, l1b_ref, l2w_ref, l2b_ref, sig_ref, z_ref, p2_ref):
    i = pl.program_id(0)
    dinv = lax.rsqrt(d0_ref[...] + d1_ref[...] + 1.0)
    h = jnp.maximum(
        dinv * (r0_ref[:, :DIM_H] + r1_ref[:, :DIM_H] + xs1_ref[:, :DIM_H])
        + b1_ref[...], 0.0)

    @pl.when(i == 0)
    def _():
        p2_ref[...] = jnp.zeros_like(p2_ref)

    _pool_accumulate(p2_ref, h, bt_ref)

    @pl.when(i == NBLK - 1)
    def _():
        s1, m1, c1 = p1_ref[0], p1_ref[1], p1_ref[2]
        s2, m2, c2 = p2_ref[0], p2_ref[1], p2_ref[2]
        mean1 = s1 / jnp.maximum(c1, 1.0)
        mean2 = s2 / jnp.maximum(c2, 1.0)
        hc = jnp.concatenate([s1, m1, mean1, s2, m2, mean2], axis=1)
        z1 = jnp.maximum(
            jnp.dot(hc, l1w_ref[...], preferred_element_type=jnp.float32)
            + l1b_ref[...], 0.0)
        z2 = jnp.dot(z1, l2w_ref[...],
                     preferred_element_type=jnp.float32) + l2b_ref[...]
        z_ref[...] = z2
        sig_ref[...] = 1.0 / (1.0 + jnp.exp(-z2))


def _k3(r0, r1, xs1, d0, d1, b1, bt, P1, l1w, l1b, l2w, l2b):
    RD = DIM_H * 2 * 3
    return pl.pallas_call(
        _k3_body,
        grid=(NBLK,),
        in_specs=[
            pl.BlockSpec((BLK, FP), lambda i: (i, 0)),
            pl.BlockSpec((BLK, FP), lambda i: (i, 0)),
            pl.BlockSpec((BLK, FP), lambda i: (i, 0)),
            pl.BlockSpec((BLK, 1), lambda i: (i, 0)),
            pl.BlockSpec((BLK, 1), lambda i: (i, 0)),
            pl.BlockSpec((1, DIM_H), lambda i: (0, 0)),
            pl.BlockSpec((1, BLK, 1), lambda i: (i, 0, 0)),
            pl.BlockSpec((3, G, DIM_H), lambda i: (0, 0, 0)),
            pl.BlockSpec((RD, RD), lambda i: (0, 0)),
            pl.BlockSpec((1, RD), lambda i: (0, 0)),
            pl.BlockSpec((RD, 1), lambda i: (0, 0)),
            pl.BlockSpec((1, 1), lambda i: (0, 0)),
        ],
        out_specs=[
            pl.BlockSpec((G, 1), lambda i: (0, 0)),
            pl.BlockSpec((G, 1), lambda i: (0, 0)),
        ],
        out_shape=[
            jax.ShapeDtypeStruct((G, 1), jnp.float32),
            jax.ShapeDtypeStruct((G, 1), jnp.float32),
        ],
        scratch_shapes=[pltpu.VMEM((3, G, DIM_H), jnp.float32)],
    )(r0, r1, xs1, d0, d1, b1, bt, P1, l1w, l1b, l2w, l2b)


def kernel(x, edge_index, batch, W0, b0, W1, b1, lin1_W, lin1_b, lin2_W,
           lin2_b):
    row = edge_index[0].reshape(NW, EPW)
    col = edge_index[1].reshape(NW, EPW)
    rowm = row[:, :NWIN * WIN].reshape(NW, NWIN, WIN)
    rowt = row[:, NWIN * WIN:].reshape(NW, 1, TAIL)
    colm = col[:, :NWIN * WIN].reshape(NW, NWIN, WIN)
    colt = col[:, NWIN * WIN:].reshape(NW, 1, TAIL)
    dcolm = col[:, :DNWIN * DWIN].reshape(NW, DNWIN, DWIN)
    dcolt = col[:, DNWIN * DWIN:].reshape(NW, 1, DTAIL)

    onesm = jnp.ones((DWIN,), jnp.float32)
    zeros1 = jnp.zeros((RPT,), jnp.float32)
    zeros64 = jnp.zeros((WIN, FP), jnp.float32)
    bt = batch.reshape(NBLK, BLK, 1)

    d0, d1 = _deg(dcolm, dcolt, onesm, zeros1)
    d0 = d0.reshape(N, 1)
    d1 = d1.reshape(N, 1)
    xs0 = _k1(x, W0, d0, d1)
    q0, q1 = _mp(xs0, rowm, colm, rowt, colt, zeros64)
    xs1, P1 = _k2(q0, q1, xs0, d0, d1, b0.reshape(1, DIM_H), W1, bt)
    r0, r1 = _mp(xs1, rowm, colm, rowt, colt, zeros64)
    sig, z = _k3(r0, r1, xs1, d0, d1, b1.reshape(1, DIM_H), bt, P1,
                 lin1_W, lin1_b.reshape(1, -1), lin2_W, lin2_b.reshape(1, 1))
    return (sig, z)
